# baseline (speedup 1.0000x reference)
import functools

import jax
import jax.numpy as jnp
from jax import lax
from jax.experimental import pallas as pl
from jax.experimental.pallas import tpu as pltpu
from jax.experimental.pallas import tpu_sc as plsc

VOCAB = 100000
NUM_SPECIAL = 3
DIM = 4096
LANES = 16
NC, NS = 2, 16
NW = NC * NS
BATCH = 1024
SEQ = 20
SEQ_PAD = 24
NB_W = BATCH // NW
PER_W = NB_W * SEQ
CHUNKS = ((0, 16), (16, 4))
NBUF = 2
BUFROWS = 16
TOKPAD = NB_W * SEQ_PAD + LANES


def _body(tok_hbm, table_hbm, embed_hbm, out_hbm,
          tok_v, tok_p, emb_v, buf0, buf1,
          gsem0, gsem1, ssem0, ssem1):
    wid = lax.axis_index("s") * NC + lax.axis_index("c")
    base = wid * PER_W
    batch0 = wid * NB_W

    pltpu.sync_copy(tok_hbm.at[pl.ds(base, PER_W)], tok_v)
    pltpu.sync_copy(embed_hbm, emb_v)

    lane = lax.iota(jnp.int32, LANES)

    def zero_step(i, carry):
        tok_p[pl.ds(i * LANES, LANES)] = jnp.zeros((LANES,), jnp.int32)
        return carry

    lax.fori_loop(0, TOKPAD // LANES, zero_step, 0)

    def pack_step(i, carry):
        t = i * LANES + lane
        dst = (t // SEQ) * SEQ_PAD + (t % SEQ)
        plsc.store_scatter(tok_p, [dst], tok_v[pl.ds(i * LANES, LANES)])
        return carry

    lax.fori_loop(0, PER_W // LANES, pack_step, 0)

    bufs = (buf0, buf1)
    gsems = (gsem0, gsem1)
    ssems = (ssem0, ssem1)

    def idx_ref(bi, ci):
        off, ln = CHUNKS[ci]
        return tok_p.at[pl.ds(bi * SEQ_PAD + off, ln)]

    def gather_descr(bi, ci, b):
        return (table_hbm.at[idx_ref(bi, ci)], bufs[b], gsems[b])

    def store_descr(bi, ci, b):
        off, ln = CHUNKS[ci]
        return (bufs[b], out_hbm.at[batch0 + bi, pl.ds(off, ln)], ssems[b])

    def start_gather(bi, ci, b):
        src, dst, sem = gather_descr(bi, ci, b)
        pltpu.async_copy(src, dst, sem)

    def wait_gather(bi, ci, b):
        src, dst, sem = gather_descr(bi, ci, b)
        pltpu.make_async_copy(src, dst, sem).wait()

    def start_store(bi, ci, b):
        src, dst, sem = store_descr(bi, ci, b)
        pltpu.async_copy(src, dst, sem)

    def wait_store(bi, ci, b):
        src, dst, sem = store_descr(bi, ci, b)
        pltpu.make_async_copy(src, dst, sem).wait()

    def fixup(bi, ci, b):
        off, ln = CHUNKS[ci]
        tokw = tok_p[pl.ds(bi * SEQ_PAD + off, LANES)]
        spec = (tokw >= VOCAB) & (lane < ln)
        any_spec = jnp.max(spec.astype(jnp.int32))

        @pl.when(any_spec > 0)
        def _():
            eidx = jnp.clip(tokw - VOCAB, 0, NUM_SPECIAL - 1)

            def col(c, carry):
                cvec = jnp.full((LANES,), 0, jnp.int32) + c
                vals = plsc.load_gather(emb_v, [eidx, cvec], mask=spec)
                plsc.store_scatter(bufs[b], [lane, cvec], vals, mask=spec)
                return carry

            lax.fori_loop(0, DIM, col, 0)

    start_gather(0, 0, 0)
    start_gather(0, 1, 1)

    def step(it, carry):
        wait_gather(it, 0, 0)
        fixup(it, 0, 0)
        start_store(it, 0, 0)

        wait_gather(it, 1, 1)
        fixup(it, 1, 1)
        start_store(it, 1, 1)

        wait_store(it, 0, 0)

        @pl.when(it < NB_W - 1)
        def _():
            start_gather(it + 1, 0, 0)

        wait_store(it, 1, 1)

        @pl.when(it < NB_W - 1)
        def _():
            start_gather(it + 1, 1, 1)

        return carry

    lax.fori_loop(0, NB_W, step, 0)


@jax.jit
def _run(tokens_flat, table, embed16):
    mesh = plsc.VectorSubcoreMesh(
        core_axis_name="c", subcore_axis_name="s",
        num_cores=NC, num_subcores=NS)
    f = pl.kernel(
        _body,
        out_type=jax.ShapeDtypeStruct((BATCH, SEQ, DIM), jnp.float32),
        mesh=mesh,
        scratch_types=[
            pltpu.VMEM((PER_W,), jnp.int32),
            pltpu.VMEM((TOKPAD,), jnp.int32),
            pltpu.VMEM((NUM_SPECIAL, DIM), jnp.float32),
            pltpu.VMEM((CHUNKS[0][1], DIM), jnp.float32),
            pltpu.VMEM((CHUNKS[1][1], DIM), jnp.float32),
            pltpu.SemaphoreType.DMA,
            pltpu.SemaphoreType.DMA,
            pltpu.SemaphoreType.DMA,
            pltpu.SemaphoreType.DMA,
        ],
        compiler_params=pltpu.CompilerParams(needs_layout_passes=False),
    )
    return f(tokens_flat, table, embed16)


def kernel(tokens, table, embed):
    embed16 = embed.astype(jnp.float16).astype(jnp.float32)
    return _run(tokens.reshape(-1), table, embed16)


# speedup vs baseline: 1.0503x; 1.0503x over previous
import functools

import jax
import jax.numpy as jnp
from jax import lax
from jax.experimental import pallas as pl
from jax.experimental.pallas import tpu as pltpu
from jax.experimental.pallas import tpu_sc as plsc

VOCAB = 100000
NUM_SPECIAL = 3
DIM = 4096
LANES = 16
NC, NS = 2, 16
NW = NC * NS
BATCH = 1024
SEQ = 20
NB_W = BATCH // NW
PER_W = NB_W * SEQ
ROWS_C = 4
NCPB = SEQ // ROWS_C
SLOT = 8
BSTRIDE = NCPB * SLOT
NBUF = NCPB
TOKPAD = NB_W * BSTRIDE + LANES


def _body(tok_hbm, table_hbm, embed_hbm, out_hbm,
          tok_v, tok_p, emb_v, buf0, buf1, buf2, buf3, buf4,
          gsem0, gsem1, gsem2, gsem3, gsem4,
          ssem0, ssem1, ssem2, ssem3, ssem4):
    wid = lax.axis_index("s") * NC + lax.axis_index("c")
    base = wid * PER_W
    batch0 = wid * NB_W

    pltpu.sync_copy(tok_hbm.at[pl.ds(base, PER_W)], tok_v)
    pltpu.sync_copy(embed_hbm, emb_v)

    lane = lax.iota(jnp.int32, LANES)

    def zero_step(i, carry):
        tok_p[pl.ds(i * LANES, LANES)] = jnp.zeros((LANES,), jnp.int32)
        return carry

    lax.fori_loop(0, TOKPAD // LANES, zero_step, 0)

    def pack_step(i, carry):
        t = i * LANES + lane
        r = t % SEQ
        dst = (t // SEQ) * BSTRIDE + (r // ROWS_C) * SLOT + (r % ROWS_C)
        plsc.store_scatter(tok_p, [dst], tok_v[pl.ds(i * LANES, LANES)])
        return carry

    lax.fori_loop(0, PER_W // LANES, pack_step, 0)

    bufs = (buf0, buf1, buf2, buf3, buf4)
    gsems = (gsem0, gsem1, gsem2, gsem3, gsem4)
    ssems = (ssem0, ssem1, ssem2, ssem3, ssem4)

    def idx_ref(bi, ci):
        return tok_p.at[pl.ds(bi * BSTRIDE + ci * SLOT, ROWS_C)]

    def gather_descr(bi, ci):
        return (table_hbm.at[idx_ref(bi, ci)], bufs[ci], gsems[ci])

    def store_descr(bi, ci):
        return (bufs[ci], out_hbm.at[batch0 + bi, pl.ds(ci * ROWS_C, ROWS_C)],
                ssems[ci])

    def start_gather(bi, ci):
        src, dst, sem = gather_descr(bi, ci)
        pltpu.async_copy(src, dst, sem)

    def wait_gather(bi, ci):
        src, dst, sem = gather_descr(bi, ci)
        pltpu.make_async_copy(src, dst, sem).wait()

    def start_store(bi, ci):
        src, dst, sem = store_descr(bi, ci)
        pltpu.async_copy(src, dst, sem)

    def wait_store(bi, ci):
        src, dst, sem = store_descr(bi, ci)
        pltpu.make_async_copy(src, dst, sem).wait()

    def fixup(bi, ci):
        tokw = tok_p[pl.ds(bi * BSTRIDE + ci * SLOT, LANES)]
        spec = (tokw >= VOCAB) & (lane < ROWS_C)
        any_spec = jnp.max(spec.astype(jnp.int32))

        @pl.when(any_spec > 0)
        def _():
            eidx = jnp.clip(tokw - VOCAB, 0, NUM_SPECIAL - 1)

            def col(c, carry):
                cvec = jnp.full((LANES,), 0, jnp.int32) + c
                vals = plsc.load_gather(emb_v, [eidx, cvec], mask=spec)
                plsc.store_scatter(bufs[ci], [lane, cvec], vals, mask=spec)
                return carry

            lax.fori_loop(0, DIM, col, 0)

    for ci in range(NBUF - 1):
        start_gather(0, ci)

    def step(it, carry):
        for ci in range(NCPB):
            wait_gather(it, ci)
            fixup(it, ci)
            start_store(it, ci)

            pci = ci - 1 if ci > 0 else NCPB - 1
            pit = it if ci > 0 else it - 1
            if ci == 0:
                @pl.when(it > 0)
                def _():
                    wait_store(pit, pci)
            else:
                wait_store(pit, pci)

            carry_b = (ci + NBUF - 1) // NCPB
            nci = (ci + NBUF - 1) % NCPB
            if carry_b == 0:
                start_gather(it, nci)
            else:
                @pl.when(it < NB_W - 1)
                def _():
                    start_gather(it + 1, nci)

        return carry

    lax.fori_loop(0, NB_W, step, 0)

    wait_store(NB_W - 1, NCPB - 1)


@jax.jit
def _run(tokens_flat, table, embed16):
    mesh = plsc.VectorSubcoreMesh(
        core_axis_name="c", subcore_axis_name="s",
        num_cores=NC, num_subcores=NS)
    f = pl.kernel(
        _body,
        out_type=jax.ShapeDtypeStruct((BATCH, SEQ, DIM), jnp.float32),
        mesh=mesh,
        scratch_types=[
            pltpu.VMEM((PER_W,), jnp.int32),
            pltpu.VMEM((TOKPAD,), jnp.int32),
            pltpu.VMEM((NUM_SPECIAL, DIM), jnp.float32),
        ] + [pltpu.VMEM((ROWS_C, DIM), jnp.float32)] * NBUF
          + [pltpu.SemaphoreType.DMA] * (2 * NBUF),
        compiler_params=pltpu.CompilerParams(needs_layout_passes=False),
    )
    return f(tokens_flat, table, embed16)


def kernel(tokens, table, embed):
    embed16 = embed.astype(jnp.float16).astype(jnp.float32)
    return _run(tokens.reshape(-1), table, embed16)
